# Initial kernel scaffold; baseline (speedup 1.0000x reference)
import jax
import jax.numpy as jnp
from jax.experimental import pallas as pl


def kernel(features, src_locs, tar_loc, src_masks, linear):
    raise NotImplementedError("write your pallas kernel here")


# speedup vs baseline: 1.1414x; 1.1414x over previous
import jax
import jax.numpy as jnp
from jax.experimental import pallas as pl


def _aidw_body(src_ref, tar_ref, mask_ref, feat_ref, lin_ref, out_ref):
    diff = src_ref[0] - tar_ref[0]
    d2 = jnp.sum(diff * diff, axis=0, keepdims=True)
    sc = jnp.where(mask_ref[0] != 0.0, 1.0 / d2, 0.0)
    w = sc / jnp.sum(sc)
    out_ref[0] = jnp.dot(feat_ref[0] * w, lin_ref[...],
                         preferred_element_type=jnp.float32)


def kernel(features, src_locs, tar_loc, src_masks, linear):
    B, L, S = features.shape
    O = linear.shape[1]
    src_t = jnp.transpose(src_locs, (0, 2, 1))
    tar_b = tar_loc[:, :, None]
    mask_f = src_masks.astype(jnp.float32)[:, None, :]

    return pl.pallas_call(
        _aidw_body,
        grid=(B,),
        in_specs=[
            pl.BlockSpec((1, 2, S), lambda b: (b, 0, 0)),
            pl.BlockSpec((1, 2, 1), lambda b: (b, 0, 0)),
            pl.BlockSpec((1, 1, S), lambda b: (b, 0, 0)),
            pl.BlockSpec((1, L, S), lambda b: (b, 0, 0)),
            pl.BlockSpec((S, O), lambda b: (0, 0)),
        ],
        out_specs=pl.BlockSpec((1, L, O), lambda b: (b, 0, 0)),
        out_shape=jax.ShapeDtypeStruct((B, L, O), jnp.float32),
    )(src_t, tar_b, mask_f, features, linear)
